# baseline (speedup 1.0000x reference)
import functools
import math

import jax
import jax.numpy as jnp
from jax import lax
from jax.experimental import pallas as pl
from jax.experimental.pallas import tpu as pltpu
from jax.experimental.pallas import tpu_sc as plsc

_S = 64.0
_M = 0.5
_SCOS = _S * math.cos(_M)
_SSIN = _S * math.sin(_M)
_BR = 3072

_NC = 2
_NS = 16
_NW = _NC * _NS


def _sc_fix_body(ct_hbm, lbl_hbm, fix_hbm, idx_v, rows_v, fx_v, sem):
    n = lbl_hbm.shape[0]
    bpw = n // _NW
    wid = lax.axis_index("s") * _NC + lax.axis_index("c")
    base = wid * bpw
    pltpu.sync_copy(lbl_hbm.at[pl.ds(base, bpw)], idx_v)
    for k in range(bpw // 16):
        l = idx_v[pl.ds(16 * k, 16)]
        idx_v[pl.ds(16 * k, 16)] = jnp.maximum(l, 0)
    pltpu.async_copy(ct_hbm.at[idx_v], rows_v, sem).wait()
    lane = lax.iota(jnp.int32, 16)
    for h in range(bpw // 16):
        c0 = base + 16 * h
        x = jnp.zeros((16,), jnp.float32)
        for k in range(16):
            v = rows_v[16 * h + k, pl.ds(c0, 16)]
            x = jnp.where(lane == k, v, x)
        y = 1.0 - x * x
        s = 0.5 * (1.0 + y)
        for _ in range(18):
            s = 0.5 * (s + y / s)
        fx_v[pl.ds(16 * h, 16)] = x * _SCOS - s * _SSIN
    pltpu.sync_copy(fx_v, fix_hbm.at[pl.ds(base, bpw)])


def _tc_block(lbl_ref, fix_ref, x_ref, o_ref):
    i = pl.program_id(0)
    x = x_ref[...]
    lbl = lbl_ref[...] - i * _BR
    row = jax.lax.broadcasted_iota(jnp.int32, x.shape, 0)
    o_ref[...] = jnp.where(row == lbl, fix_ref[...], x * _S)


def kernel(cosine, label):
    n, c = cosine.shape
    ct = cosine.T
    lbl = label.astype(jnp.int32)
    bpw = n // _NW

    sc_fix = functools.partial(
        pl.kernel,
        out_type=jax.ShapeDtypeStruct((n,), jnp.float32),
        mesh=plsc.VectorSubcoreMesh(core_axis_name="c", subcore_axis_name="s"),
        scratch_types=[
            pltpu.VMEM((bpw,), jnp.int32),
            pltpu.VMEM((bpw, n), jnp.float32),
            pltpu.VMEM((bpw,), jnp.float32),
            pltpu.SemaphoreType.DMA,
        ],
    )(_sc_fix_body)
    fix = sc_fix(ct, lbl)

    out_t = pl.pallas_call(
        _tc_block,
        grid=(pl.cdiv(c, _BR),),
        in_specs=[
            pl.BlockSpec((1, n), lambda i: (0, 0)),
            pl.BlockSpec((1, n), lambda i: (0, 0)),
            pl.BlockSpec((_BR, n), lambda i: (i, 0)),
        ],
        out_specs=pl.BlockSpec((_BR, n), lambda i: (i, 0)),
        out_shape=jax.ShapeDtypeStruct((c, n), cosine.dtype),
    )(lbl.reshape(1, n), fix.reshape(1, n), ct)
    return out_t.T


# speedup vs baseline: 1.0155x; 1.0155x over previous
import functools
import math

import jax
import jax.numpy as jnp
from jax import lax
from jax.experimental import pallas as pl
from jax.experimental.pallas import tpu as pltpu
from jax.experimental.pallas import tpu_sc as plsc

_S = 64.0
_M = 0.5
_SCOS = _S * math.cos(_M)
_SSIN = _S * math.sin(_M)
_BR = 3072
_SPLIT = 8 * _BR

_NC = 2
_NS = 16
_NW = _NC * _NS


def _sc_fix_body(ct_hbm, lbl_hbm, fix_hbm, idx_v, rows_v, fx_v, sem):
    n = lbl_hbm.shape[0]
    bpw = n // _NW
    wid = lax.axis_index("s") * _NC + lax.axis_index("c")
    base = wid * bpw
    pltpu.sync_copy(lbl_hbm.at[pl.ds(base, bpw)], idx_v)
    for k in range(bpw // 16):
        l = idx_v[pl.ds(16 * k, 16)]
        idx_v[pl.ds(16 * k, 16)] = jnp.maximum(l, 0)
    pltpu.async_copy(ct_hbm.at[idx_v], rows_v, sem).wait()
    lane = lax.iota(jnp.int32, 16)
    for h in range(bpw // 16):
        c0 = base + 16 * h
        x = jnp.zeros((16,), jnp.float32)
        for k in range(16):
            v = rows_v[16 * h + k, pl.ds(c0, 16)]
            x = jnp.where(lane == k, v, x)
        y = 1.0 - x * x
        s = 0.5 * (1.0 + y)
        for _ in range(18):
            s = 0.5 * (s + y / s)
        fx_v[pl.ds(16 * h, 16)] = x * _SCOS - s * _SSIN
    pltpu.sync_copy(fx_v, fix_hbm.at[pl.ds(base, bpw)])


def _tc_dense_block(lbl_ref, x_ref, o_ref):
    i = pl.program_id(0) + _SPLIT // _BR
    x = x_ref[...]
    lbl = lbl_ref[...] - i * _BR
    row = jax.lax.broadcasted_iota(jnp.int32, x.shape, 0)
    y = 1.0 - x * x
    fix = x * _SCOS - (y * jax.lax.rsqrt(y)) * _SSIN
    o_ref[...] = jnp.where(row == lbl, fix, x * _S)


def _tc_fix_block(lbl_ref, fix_ref, x_ref, alias_ref, o_ref):
    del alias_ref
    i = pl.program_id(0)
    x = x_ref[...]
    lbl = lbl_ref[...] - i * _BR
    row = jax.lax.broadcasted_iota(jnp.int32, x.shape, 0)
    o_ref[...] = jnp.where(row == lbl, fix_ref[...], x * _S)


def kernel(cosine, label):
    n, c = cosine.shape
    ct = cosine.T
    lbl = label.astype(jnp.int32)
    lbl2 = lbl.reshape(1, n)
    bpw = n // _NW

    sc_fix = functools.partial(
        pl.kernel,
        out_type=jax.ShapeDtypeStruct((n,), jnp.float32),
        mesh=plsc.VectorSubcoreMesh(core_axis_name="c", subcore_axis_name="s"),
        scratch_types=[
            pltpu.VMEM((bpw,), jnp.int32),
            pltpu.VMEM((bpw, n), jnp.float32),
            pltpu.VMEM((bpw,), jnp.float32),
            pltpu.SemaphoreType.DMA,
        ],
    )(_sc_fix_body)
    fix = sc_fix(ct, lbl)

    off = _SPLIT // _BR
    out_a = pl.pallas_call(
        _tc_dense_block,
        grid=(pl.cdiv(c, _BR) - off,),
        in_specs=[
            pl.BlockSpec((1, n), lambda i: (0, 0)),
            pl.BlockSpec((_BR, n), lambda i: (i + off, 0)),
        ],
        out_specs=pl.BlockSpec((_BR, n), lambda i: (i + off, 0)),
        out_shape=jax.ShapeDtypeStruct((c, n), cosine.dtype),
    )(lbl2, ct)

    out_t = pl.pallas_call(
        _tc_fix_block,
        grid=(off,),
        in_specs=[
            pl.BlockSpec((1, n), lambda i: (0, 0)),
            pl.BlockSpec((1, n), lambda i: (0, 0)),
            pl.BlockSpec((_BR, n), lambda i: (i, 0)),
            pl.BlockSpec(memory_space=pl.ANY),
        ],
        out_specs=pl.BlockSpec((_BR, n), lambda i: (i, 0)),
        out_shape=jax.ShapeDtypeStruct((c, n), cosine.dtype),
        input_output_aliases={3: 0},
    )(lbl2, fix.reshape(1, n), ct, out_a)
    return out_t.T
